# baseline (speedup 1.0000x reference)
import jax
import jax.numpy as jnp
from jax.experimental import pallas as pl
from jax.experimental.pallas import tpu as pltpu

NUM_EXPERTS = 64
TOP_K = 8
HIDDEN = 4096
TOKENS = 32768
BT = 1024
NH = 4

NEG_INF = float("-inf")


def _topk_softmax_t(lt):
    iota_f = jax.lax.broadcasted_iota(jnp.int32, lt.shape, 0).astype(jnp.float32)
    cur = lt
    vals = []
    idxs = []
    for _ in range(TOP_K):
        m = jnp.max(cur, axis=0, keepdims=True)
        is_max = cur == m
        idx = jnp.min(jnp.where(is_max, iota_f, 64.0), axis=0, keepdims=True)
        vals.append(m)
        idxs.append(idx)
        cur = jnp.where(iota_f == idx, NEG_INF, cur)

    v = jnp.concatenate(vals, axis=0)
    e = jnp.exp(v - v[0:1, :])
    w = e / jnp.sum(e, axis=0, keepdims=True)
    return w, jnp.concatenate(idxs, axis=0).astype(jnp.int32)


def _router_kernel(x_ref, w_ref, w_out_ref, i_out_ref):
    wmat = w_ref[...]
    rows = BT // NH
    lts = [
        jax.lax.dot_general(
            wmat, x_ref[pl.ds(h * rows, rows), :],
            dimension_numbers=(((1,), (1,)), ((), ())),
            preferred_element_type=jnp.float32,
        )
        for h in range(NH)
    ]
    for h in range(NH):
        w, i = _topk_softmax_t(lts[h])
        w_out_ref[:, pl.ds(h * rows, rows)] = w
        i_out_ref[:, pl.ds(h * rows, rows)] = i


@jax.jit
def kernel(hidden_states, W):
    grid = (TOKENS // BT,)
    out_w, out_i = pl.pallas_call(
        _router_kernel,
        grid=grid,
        in_specs=[
            pl.BlockSpec((BT, HIDDEN), lambda i: (i, 0)),
            pl.BlockSpec((NUM_EXPERTS, HIDDEN), lambda i: (0, 0)),
        ],
        out_specs=[
            pl.BlockSpec((TOP_K, BT), lambda i: (0, i)),
            pl.BlockSpec((TOP_K, BT), lambda i: (0, i)),
        ],
        out_shape=[
            jax.ShapeDtypeStruct((TOP_K, TOKENS), jnp.float32),
            jax.ShapeDtypeStruct((TOP_K, TOKENS), jnp.int32),
        ],
        compiler_params=pltpu.CompilerParams(
            dimension_semantics=("parallel",),
        ),
    )(hidden_states, W)
    return (out_w.T, out_i.T)
